# Initial kernel scaffold; baseline (speedup 1.0000x reference)
import jax
import jax.numpy as jnp
from jax.experimental import pallas as pl


def kernel(features, labels):
    raise NotImplementedError("write your pallas kernel here")


# speedup vs baseline: 2.6066x; 2.6066x over previous
import functools

import jax
import jax.numpy as jnp
from jax.experimental import pallas as pl
from jax.experimental.pallas import tpu as pltpu

_TEMPERATURE = 0.07
_NUM_HARD = 5
_EPS = 1e-08
_NEG_INF = -3.0e38


def _supcon_block_kernel(f_ref, ft_ref, lrow_ref, lcol_ref, out_ref, *, blk_r, batch):
    i = pl.program_id(0)

    fb = f_ref[pl.ds(i * blk_r, blk_r), :]
    nb = jnp.sqrt(jnp.sum(fb * fb, axis=1, keepdims=True))
    fbn = fb / jnp.maximum(nb, 1e-12)

    ft = ft_ref[:, :]
    nt = jnp.sqrt(jnp.sum(ft * ft, axis=0, keepdims=True))
    ftn = ft / jnp.maximum(nt, 1e-12)

    g = jax.lax.dot_general(
        fbn, ftn, (((1,), (0,)), ((), ())),
        preferred_element_type=jnp.float32,
        precision=jax.lax.Precision.HIGHEST,
    )
    sim = jnp.clip(g, -10.0, 10.0) * (1.0 / _TEMPERATURE)

    lrow = lrow_ref[:, :]
    lcol = lcol_ref[:, :]
    eq = lrow == lcol
    eqf = jnp.where(eq, 1.0, 0.0)
    cidx = jax.lax.broadcasted_iota(jnp.int32, (blk_r, batch), 1)
    ridx = jax.lax.broadcasted_iota(jnp.int32, (blk_r, batch), 0) + i * blk_r
    posf = jnp.where(eq & (cidx != ridx), 1.0, 0.0)

    m = jnp.max(sim, axis=1, keepdims=True)
    e = jnp.exp(sim - m)
    s_all = jnp.sum(e, axis=1, keepdims=True)
    s_same = jnp.sum(e * eqf, axis=1, keepdims=True)
    cnt_same = jnp.sum(eqf, axis=1, keepdims=True)
    exp_pos = jnp.sum(e * posf, axis=1, keepdims=True)
    num_neg = jnp.float32(batch) - cnt_same
    s_allneg = (s_all - s_same) + cnt_same * jnp.exp(-m)

    ns = jnp.where(eq, 0.0, sim)
    vals = []
    cnts = []
    work = ns
    for _ in range(_NUM_HARD):
        mt = jnp.max(work, axis=1, keepdims=True)
        hit = work == mt
        ct = jnp.sum(jnp.where(hit, 1.0, 0.0), axis=1, keepdims=True)
        work = jnp.where(hit, _NEG_INF, work)
        vals.append(mt)
        cnts.append(ct)

    exp_neg_m = jnp.exp(-m)
    s_hard = jnp.zeros_like(m)
    e_hard = []
    prev = jnp.zeros_like(m)
    slot_vals = []
    for s in range(_NUM_HARD):
        sv = jnp.zeros_like(m)
        for t in range(_NUM_HARD):
            if t == 0:
                lo = jnp.zeros_like(m)
            else:
                lo = cnts[0]
                for u in range(1, t):
                    lo = lo + cnts[u]
            take = (lo <= jnp.float32(s)) & (jnp.float32(s) < lo + cnts[t])
            sv = jnp.where(take, vals[t], sv)
        sv = jnp.where(jnp.float32(s) < jnp.minimum(num_neg, jnp.float32(_NUM_HARD)),
                       sv, 0.0)
        slot_vals.append(sv)

    for s in range(_NUM_HARD):
        eh = jnp.exp(slot_vals[s] - m)
        e_hard.append(eh)
        s_hard = s_hard + eh

    loss_blk = jnp.zeros((), jnp.float32)
    for s in range(_NUM_HARD):
        denom = exp_pos + s_hard + s_allneg - e_hard[s]
        lp = jnp.log(exp_pos / (denom + _EPS) + _EPS)
        loss_blk = loss_blk + jnp.sum(lp)

    @pl.when(i == 0)
    def _():
        out_ref[:, :] = jnp.zeros((1, 1), jnp.float32)

    out_ref[:, :] += loss_blk[None, None]


@jax.jit
def kernel(features, labels):
    batch, dim = features.shape
    blk_r = 256
    grid = batch // blk_r

    ft = features.T
    lrow = labels.reshape(batch, 1)
    lcol = labels.reshape(1, batch)

    body = functools.partial(_supcon_block_kernel, blk_r=blk_r, batch=batch)
    total = pl.pallas_call(
        body,
        grid=(grid,),
        in_specs=[
            pl.BlockSpec((batch, dim), lambda i: (0, 0)),
            pl.BlockSpec((dim, batch), lambda i: (0, 0)),
            pl.BlockSpec((blk_r, 1), lambda i: (i, 0)),
            pl.BlockSpec((1, batch), lambda i: (0, 0)),
        ],
        out_specs=pl.BlockSpec((1, 1), lambda i: (0, 0)),
        out_shape=jax.ShapeDtypeStruct((1, 1), jnp.float32),
    )(features, ft, lrow, lcol)

    return -total[0, 0] / jnp.float32(batch * _NUM_HARD)


# speedup vs baseline: 3.2158x; 1.2337x over previous
import functools

import jax
import jax.numpy as jnp
from jax.experimental import pallas as pl
from jax.experimental.pallas import tpu as pltpu

_TEMPERATURE = 0.07
_NUM_HARD = 5
_EPS = 1e-08
_NEG_INF = -3.0e38


def _supcon_block_kernel(f_ref, ft_ref, lrow_ref, lcol_ref, out_ref, *, blk_r, batch):
    i = pl.program_id(0)

    fb = f_ref[pl.ds(i * blk_r, blk_r), :]
    nb = jnp.sqrt(jnp.sum(fb * fb, axis=1, keepdims=True))
    fbn = fb / jnp.maximum(nb, 1e-12)

    ft = ft_ref[:, :]
    nt = jnp.sqrt(jnp.sum(ft * ft, axis=0, keepdims=True))
    ftn = ft / jnp.maximum(nt, 1e-12)

    g = jax.lax.dot_general(
        fbn, ftn, (((1,), (0,)), ((), ())),
        preferred_element_type=jnp.float32,
    )
    sim = jnp.clip(g, -10.0, 10.0) * (1.0 / _TEMPERATURE)

    lrow = lrow_ref[:, :]
    lcol = lcol_ref[:, :]
    eq = lrow == lcol
    eqf = jnp.where(eq, 1.0, 0.0)

    m = jnp.max(sim, axis=1, keepdims=True)
    e = jnp.exp(sim - m)
    s_all = jnp.sum(e, axis=1, keepdims=True)
    s_same = jnp.sum(e * eqf, axis=1, keepdims=True)
    cnt_same = jnp.sum(eqf, axis=1, keepdims=True)
    cidx = jax.lax.broadcasted_iota(jnp.int32, (blk_r, batch), 1)
    ridx = jax.lax.broadcasted_iota(jnp.int32, (blk_r, batch), 0) + i * blk_r
    posf = jnp.where(eq & (cidx != ridx), 1.0, 0.0)
    exp_pos = jnp.sum(e * posf, axis=1, keepdims=True)
    num_neg = jnp.float32(batch) - cnt_same
    s_allneg = (s_all - s_same) + cnt_same * jnp.exp(-m)

    ns = jnp.where(eq, 0.0, sim)
    vals = []
    cnts = []
    work = ns
    for _ in range(_NUM_HARD):
        mt = jnp.max(work, axis=1, keepdims=True)
        hit = work == mt
        ct = jnp.sum(jnp.where(hit, 1.0, 0.0), axis=1, keepdims=True)
        work = jnp.where(hit, _NEG_INF, work)
        vals.append(mt)
        cnts.append(ct)

    exp_neg_m = jnp.exp(-m)
    s_hard = jnp.zeros_like(m)
    e_hard = []
    prev = jnp.zeros_like(m)
    slot_vals = []
    for s in range(_NUM_HARD):
        sv = jnp.zeros_like(m)
        for t in range(_NUM_HARD):
            if t == 0:
                lo = jnp.zeros_like(m)
            else:
                lo = cnts[0]
                for u in range(1, t):
                    lo = lo + cnts[u]
            take = (lo <= jnp.float32(s)) & (jnp.float32(s) < lo + cnts[t])
            sv = jnp.where(take, vals[t], sv)
        sv = jnp.where(jnp.float32(s) < jnp.minimum(num_neg, jnp.float32(_NUM_HARD)),
                       sv, 0.0)
        slot_vals.append(sv)

    for s in range(_NUM_HARD):
        eh = jnp.exp(slot_vals[s] - m)
        e_hard.append(eh)
        s_hard = s_hard + eh

    loss_blk = jnp.zeros((), jnp.float32)
    for s in range(_NUM_HARD):
        denom = exp_pos + s_hard + s_allneg - e_hard[s]
        lp = jnp.log(exp_pos / (denom + _EPS) + _EPS)
        loss_blk = loss_blk + jnp.sum(lp)

    @pl.when(i == 0)
    def _():
        out_ref[:, :] = jnp.zeros((1, 1), jnp.float32)

    out_ref[:, :] += loss_blk[None, None]


@jax.jit
def kernel(features, labels):
    batch, dim = features.shape
    blk_r = 256
    grid = batch // blk_r

    ft = features.T
    lrow = labels.reshape(batch, 1)
    lcol = labels.reshape(1, batch)

    body = functools.partial(_supcon_block_kernel, blk_r=blk_r, batch=batch)
    total = pl.pallas_call(
        body,
        grid=(grid,),
        in_specs=[
            pl.BlockSpec((batch, dim), lambda i: (0, 0)),
            pl.BlockSpec((dim, batch), lambda i: (0, 0)),
            pl.BlockSpec((blk_r, 1), lambda i: (i, 0)),
            pl.BlockSpec((1, batch), lambda i: (0, 0)),
        ],
        out_specs=pl.BlockSpec((1, 1), lambda i: (0, 0)),
        out_shape=jax.ShapeDtypeStruct((1, 1), jnp.float32),
    )(features, ft, lrow, lcol)

    return -total[0, 0] / jnp.float32(batch * _NUM_HARD)


# speedup vs baseline: 3.4314x; 1.0670x over previous
import functools

import jax
import jax.numpy as jnp
from jax.experimental import pallas as pl
from jax.experimental.pallas import tpu as pltpu

_TEMPERATURE = 0.07
_NUM_HARD = 5
_EPS = 1e-08
_NEG_INF = -3.0e38


def _supcon_block_kernel(f_ref, ft_ref, lrow_ref, lcol_ref, out_ref, *, blk_r, batch):
    i = pl.program_id(0)

    fb = f_ref[pl.ds(i * blk_r, blk_r), :]
    nb = jnp.sqrt(jnp.sum(fb * fb, axis=1, keepdims=True))
    fbn = fb / jnp.maximum(nb, 1e-12)

    ft = ft_ref[:, :]
    nt = jnp.sqrt(jnp.sum(ft * ft, axis=0, keepdims=True))
    ftn = ft / jnp.maximum(nt, 1e-12)

    g = jax.lax.dot_general(
        fbn, ftn, (((1,), (0,)), ((), ())),
        preferred_element_type=jnp.float32,
    )
    sim = jnp.clip(g, -10.0, 10.0) * (1.0 / _TEMPERATURE)

    lrow = lrow_ref[:, :]
    lcol = lcol_ref[:, :]
    eq = lrow == lcol

    m = jnp.max(sim, axis=1, keepdims=True)
    e = jnp.exp(sim - m)
    s_all = jnp.sum(e, axis=1, keepdims=True)
    cidx = jax.lax.broadcasted_iota(jnp.int32, (blk_r, batch), 1)
    ridx = jax.lax.broadcasted_iota(jnp.int32, (blk_r, batch), 0) + i * blk_r
    posf = jnp.where(eq & (cidx != ridx), 1.0, 0.0)
    exp_pos = jnp.sum(e * posf, axis=1, keepdims=True)
    cnt_same = jnp.sum(posf, axis=1, keepdims=True) + 1.0
    sim_d = jnp.clip(jnp.sum(fbn * fbn, axis=1, keepdims=True),
                     -10.0, 10.0) * (1.0 / _TEMPERATURE)
    e_diag = jnp.exp(sim_d - m)
    s_same = exp_pos + e_diag
    num_neg = jnp.float32(batch) - cnt_same
    s_allneg = (s_all - s_same) + cnt_same * jnp.exp(-m)

    ns = jnp.where(eq, 0.0, sim)
    vals = []
    cnts = []
    work = ns
    for _ in range(_NUM_HARD):
        mt = jnp.max(work, axis=1, keepdims=True)
        hit = work == mt
        ct = jnp.sum(jnp.where(hit, 1.0, 0.0), axis=1, keepdims=True)
        work = jnp.where(hit, _NEG_INF, work)
        vals.append(mt)
        cnts.append(ct)

    exp_neg_m = jnp.exp(-m)
    s_hard = jnp.zeros_like(m)
    e_hard = []
    prev = jnp.zeros_like(m)
    slot_vals = []
    for s in range(_NUM_HARD):
        sv = jnp.zeros_like(m)
        for t in range(_NUM_HARD):
            if t == 0:
                lo = jnp.zeros_like(m)
            else:
                lo = cnts[0]
                for u in range(1, t):
                    lo = lo + cnts[u]
            take = (lo <= jnp.float32(s)) & (jnp.float32(s) < lo + cnts[t])
            sv = jnp.where(take, vals[t], sv)
        sv = jnp.where(jnp.float32(s) < jnp.minimum(num_neg, jnp.float32(_NUM_HARD)),
                       sv, 0.0)
        slot_vals.append(sv)

    for s in range(_NUM_HARD):
        eh = jnp.exp(slot_vals[s] - m)
        e_hard.append(eh)
        s_hard = s_hard + eh

    loss_blk = jnp.zeros((), jnp.float32)
    for s in range(_NUM_HARD):
        denom = exp_pos + s_hard + s_allneg - e_hard[s]
        lp = jnp.log(exp_pos / (denom + _EPS) + _EPS)
        loss_blk = loss_blk + jnp.sum(lp)

    @pl.when(i == 0)
    def _():
        out_ref[:, :] = jnp.zeros((1, 1), jnp.float32)

    out_ref[:, :] += loss_blk[None, None]


@jax.jit
def kernel(features, labels):
    batch, dim = features.shape
    blk_r = 256
    grid = batch // blk_r

    ft = features.T
    lrow = labels.reshape(batch, 1)
    lcol = labels.reshape(1, batch)

    body = functools.partial(_supcon_block_kernel, blk_r=blk_r, batch=batch)
    total = pl.pallas_call(
        body,
        grid=(grid,),
        in_specs=[
            pl.BlockSpec((batch, dim), lambda i: (0, 0)),
            pl.BlockSpec((dim, batch), lambda i: (0, 0)),
            pl.BlockSpec((blk_r, 1), lambda i: (i, 0)),
            pl.BlockSpec((1, batch), lambda i: (0, 0)),
        ],
        out_specs=pl.BlockSpec((1, 1), lambda i: (0, 0)),
        out_shape=jax.ShapeDtypeStruct((1, 1), jnp.float32),
    )(features, ft, lrow, lcol)

    return -total[0, 0] / jnp.float32(batch * _NUM_HARD)


# speedup vs baseline: 4.6132x; 1.3444x over previous
import functools

import jax
import jax.numpy as jnp
from jax.experimental import pallas as pl
from jax.experimental.pallas import tpu as pltpu

_TEMPERATURE = 0.07
_NUM_HARD = 5
_EPS = 1e-08
_NEG_INF = -3.0e38
_LANES = 128
_SUB_R = 32


def _supcon_block_kernel(f_ref, ft_ref, lrow_ref, lcol_ref, cidx_ref, out_ref,
                         *, blk_r, batch):
    i = pl.program_id(0)

    fb = f_ref[pl.ds(i * blk_r, blk_r), :]
    nb = jnp.sqrt(jnp.sum(fb * fb, axis=1, keepdims=True))
    fbn = fb / jnp.maximum(nb, 1e-12)

    ft = ft_ref[:, :]
    nt = jnp.sqrt(jnp.sum(ft * ft, axis=0, keepdims=True))
    ftn = ft / jnp.maximum(nt, 1e-12)

    g = jax.lax.dot_general(
        fbn, ftn, (((1,), (0,)), ((), ())),
        preferred_element_type=jnp.float32,
    )
    sim = jnp.clip(g, -10.0, 10.0) * (1.0 / _TEMPERATURE)

    lrow = lrow_ref[:, :]
    lcol = lcol_ref[:, :]
    eq = lrow == lcol

    m = jnp.max(sim, axis=1, keepdims=True)
    e = jnp.exp(sim - m)
    s_all = jnp.sum(e, axis=1, keepdims=True)
    ridx = jax.lax.broadcasted_iota(jnp.int32, (blk_r, 1), 0) + i * blk_r
    posf = jnp.where(eq & (cidx_ref[:, :] != ridx), 1.0, 0.0)
    exp_pos = jnp.sum(e * posf, axis=1, keepdims=True)
    cnt_same = jnp.sum(posf, axis=1, keepdims=True) + 1.0
    num_neg = jnp.float32(batch) - cnt_same
    sim_d = jnp.clip(jnp.sum(fbn * fbn, axis=1, keepdims=True),
                     -10.0, 10.0) * (1.0 / _TEMPERATURE)
    e_diag = jnp.exp(sim_d - m)
    s_same = exp_pos + e_diag
    s_allneg = (s_all - s_same) + cnt_same * jnp.exp(-m)

    n_chunks = batch // _LANES
    cand_rows = []
    for sb in range(blk_r // _SUB_R):
        r0 = sb * _SUB_R
        m1 = jnp.full((_SUB_R, _LANES), _NEG_INF)
        m2 = m1
        m3 = m1
        m4 = m1
        m5 = m1
        for c in range(n_chunks):
            c0 = c * _LANES
            v = jnp.where(eq[r0:r0 + _SUB_R, c0:c0 + _LANES], 0.0,
                          sim[r0:r0 + _SUB_R, c0:c0 + _LANES])
            t = jnp.maximum(m1, v)
            v = jnp.minimum(m1, v)
            m1 = t
            t = jnp.maximum(m2, v)
            v = jnp.minimum(m2, v)
            m2 = t
            t = jnp.maximum(m3, v)
            v = jnp.minimum(m3, v)
            m3 = t
            t = jnp.maximum(m4, v)
            v = jnp.minimum(m4, v)
            m4 = t
            m5 = jnp.maximum(m5, v)
        cand_rows.append(jnp.concatenate([m1, m2, m3, m4, m5], axis=1))
    cand = jnp.concatenate(cand_rows, axis=0)

    vals = []
    cnts = []
    work = cand
    for _ in range(_NUM_HARD):
        mt = jnp.max(work, axis=1, keepdims=True)
        hit = work == mt
        ct = jnp.sum(jnp.where(hit, 1.0, 0.0), axis=1, keepdims=True)
        work = jnp.where(hit, _NEG_INF, work)
        vals.append(mt)
        cnts.append(ct)

    s_hard = jnp.zeros_like(m)
    e_hard = []
    slot_vals = []
    for s in range(_NUM_HARD):
        sv = jnp.zeros_like(m)
        for t in range(_NUM_HARD):
            if t == 0:
                lo = jnp.zeros_like(m)
            else:
                lo = cnts[0]
                for u in range(1, t):
                    lo = lo + cnts[u]
            take = (lo <= jnp.float32(s)) & (jnp.float32(s) < lo + cnts[t])
            sv = jnp.where(take, vals[t], sv)
        sv = jnp.where(jnp.float32(s) < jnp.minimum(num_neg, jnp.float32(_NUM_HARD)),
                       sv, 0.0)
        slot_vals.append(sv)

    for s in range(_NUM_HARD):
        eh = jnp.exp(slot_vals[s] - m)
        e_hard.append(eh)
        s_hard = s_hard + eh

    loss_blk = jnp.zeros((), jnp.float32)
    for s in range(_NUM_HARD):
        denom = exp_pos + s_hard + s_allneg - e_hard[s]
        lp = jnp.log(exp_pos / (denom + _EPS) + _EPS)
        loss_blk = loss_blk + jnp.sum(lp)

    @pl.when(i == 0)
    def _():
        out_ref[:, :] = jnp.zeros((1, 1), jnp.float32)

    out_ref[:, :] += loss_blk[None, None]


@jax.jit
def kernel(features, labels):
    batch, dim = features.shape
    blk_r = 256
    grid = batch // blk_r

    ft = features.T
    lrow = labels.reshape(batch, 1)
    lcol = labels.reshape(1, batch)
    cidx = jnp.arange(batch, dtype=jnp.int32).reshape(1, batch)

    body = functools.partial(_supcon_block_kernel, blk_r=blk_r, batch=batch)
    total = pl.pallas_call(
        body,
        grid=(grid,),
        in_specs=[
            pl.BlockSpec((batch, dim), lambda i: (0, 0)),
            pl.BlockSpec((dim, batch), lambda i: (0, 0)),
            pl.BlockSpec((blk_r, 1), lambda i: (i, 0)),
            pl.BlockSpec((1, batch), lambda i: (0, 0)),
            pl.BlockSpec((1, batch), lambda i: (0, 0)),
        ],
        out_specs=pl.BlockSpec((1, 1), lambda i: (0, 0)),
        out_shape=jax.ShapeDtypeStruct((1, 1), jnp.float32),
    )(features, ft, lrow, lcol, cidx)

    return -total[0, 0] / jnp.float32(batch * _NUM_HARD)


# speedup vs baseline: 4.7542x; 1.0305x over previous
import functools

import jax
import jax.numpy as jnp
from jax.experimental import pallas as pl
from jax.experimental.pallas import tpu as pltpu

_TEMPERATURE = 0.07
_NUM_HARD = 5
_EPS = 1e-08
_NEG_INF = -3.0e38
_LANES = 128
_SUB_R = 32


def _supcon_block_kernel(f_ref, ft_ref, lrow_ref, lcol_ref, cidx_ref, out_ref,
                         *, blk_r, batch):
    i = pl.program_id(0)

    fb = f_ref[pl.ds(i * blk_r, blk_r), :]
    nb = jnp.sqrt(jnp.sum(fb * fb, axis=1, keepdims=True))
    fbn = fb / jnp.maximum(nb, 1e-12)

    ft = ft_ref[:, :]
    nt = jnp.sqrt(jnp.sum(ft * ft, axis=0, keepdims=True))
    ftn = ft / jnp.maximum(nt, 1e-12)

    g = jax.lax.dot_general(
        fbn, ftn, (((1,), (0,)), ((), ())),
        preferred_element_type=jnp.float32,
    )
    sim = jnp.clip(g, -10.0, 10.0) * (1.0 / _TEMPERATURE)

    lrow = lrow_ref[:, :]
    lcol = lcol_ref[:, :]
    eq = lrow == lcol

    m = jnp.clip(jnp.sum(fbn * fbn, axis=1, keepdims=True),
                 -10.0, 10.0) * (1.0 / _TEMPERATURE)
    e = jnp.exp(sim - m)
    s_all = jnp.sum(e, axis=1, keepdims=True)
    ridx = jax.lax.broadcasted_iota(jnp.int32, (blk_r, 1), 0) + i * blk_r
    posf = jnp.where(eq & (cidx_ref[:, :] != ridx), 1.0, 0.0)
    exp_pos = jnp.sum(e * posf, axis=1, keepdims=True)
    cnt_same = jnp.sum(posf, axis=1, keepdims=True) + 1.0
    num_neg = jnp.float32(batch) - cnt_same
    s_same = exp_pos + 1.0
    s_allneg = (s_all - s_same) + cnt_same * jnp.exp(-m)

    n_chunks = batch // _LANES
    cand_rows = []
    for sb in range(blk_r // _SUB_R):
        r0 = sb * _SUB_R
        m1 = jnp.full((_SUB_R, _LANES), _NEG_INF)
        m2 = m1
        m3 = m1
        m4 = m1
        m5 = m1
        for c in range(n_chunks):
            c0 = c * _LANES
            v = jnp.where(eq[r0:r0 + _SUB_R, c0:c0 + _LANES], 0.0,
                          sim[r0:r0 + _SUB_R, c0:c0 + _LANES])
            t = jnp.maximum(m1, v)
            v = jnp.minimum(m1, v)
            m1 = t
            t = jnp.maximum(m2, v)
            v = jnp.minimum(m2, v)
            m2 = t
            t = jnp.maximum(m3, v)
            v = jnp.minimum(m3, v)
            m3 = t
            t = jnp.maximum(m4, v)
            v = jnp.minimum(m4, v)
            m4 = t
            m5 = jnp.maximum(m5, v)
        cand_rows.append(jnp.concatenate([m1, m2, m3, m4, m5], axis=1))
    cand = jnp.concatenate(cand_rows, axis=0)

    vals = []
    cnts = []
    work = cand
    for _ in range(_NUM_HARD):
        mt = jnp.max(work, axis=1, keepdims=True)
        hit = work == mt
        ct = jnp.sum(jnp.where(hit, 1.0, 0.0), axis=1, keepdims=True)
        work = jnp.where(hit, _NEG_INF, work)
        vals.append(mt)
        cnts.append(ct)

    los = [jnp.zeros_like(m)]
    for t in range(1, _NUM_HARD):
        los.append(los[t - 1] + cnts[t - 1])
    s_hard = jnp.zeros_like(m)
    e_hard = []
    slot_vals = []
    for s in range(_NUM_HARD):
        sv = jnp.zeros_like(m)
        for t in range(_NUM_HARD):
            lo = los[t]
            take = (lo <= jnp.float32(s)) & (jnp.float32(s) < lo + cnts[t])
            sv = jnp.where(take, vals[t], sv)
        sv = jnp.where(jnp.float32(s) < jnp.minimum(num_neg, jnp.float32(_NUM_HARD)),
                       sv, 0.0)
        slot_vals.append(sv)

    for s in range(_NUM_HARD):
        eh = jnp.exp(slot_vals[s] - m)
        e_hard.append(eh)
        s_hard = s_hard + eh

    loss_blk = jnp.zeros((), jnp.float32)
    for s in range(_NUM_HARD):
        denom = exp_pos + s_hard + s_allneg - e_hard[s]
        lp = jnp.log(exp_pos / (denom + _EPS) + _EPS)
        loss_blk = loss_blk + jnp.sum(lp)

    @pl.when(i == 0)
    def _():
        out_ref[:, :] = jnp.zeros((1, 1), jnp.float32)

    out_ref[:, :] += loss_blk[None, None]


@jax.jit
def kernel(features, labels):
    batch, dim = features.shape
    blk_r = 256
    grid = batch // blk_r

    ft = features.T
    lrow = labels.reshape(batch, 1)
    lcol = labels.reshape(1, batch)
    cidx = jnp.arange(batch, dtype=jnp.int32).reshape(1, batch)

    body = functools.partial(_supcon_block_kernel, blk_r=blk_r, batch=batch)
    total = pl.pallas_call(
        body,
        grid=(grid,),
        in_specs=[
            pl.BlockSpec((batch, dim), lambda i: (0, 0)),
            pl.BlockSpec((dim, batch), lambda i: (0, 0)),
            pl.BlockSpec((blk_r, 1), lambda i: (i, 0)),
            pl.BlockSpec((1, batch), lambda i: (0, 0)),
            pl.BlockSpec((1, batch), lambda i: (0, 0)),
        ],
        out_specs=pl.BlockSpec((1, 1), lambda i: (0, 0)),
        out_shape=jax.ShapeDtypeStruct((1, 1), jnp.float32),
    )(features, ft, lrow, lcol, cidx)

    return -total[0, 0] / jnp.float32(batch * _NUM_HARD)


# speedup vs baseline: 4.7895x; 1.0074x over previous
import functools

import jax
import jax.numpy as jnp
from jax.experimental import pallas as pl
from jax.experimental.pallas import tpu as pltpu

_TEMPERATURE = 0.07
_NUM_HARD = 5
_EPS = 1e-08
_NEG_INF = -3.0e38
_LANES = 128
_SUB_R = 32


def _supcon_block_kernel(f_ref, ft_ref, lrow_ref, lcol_ref, cidx_ref, out_ref,
                         *, blk_r, batch):
    i = pl.program_id(0)

    fb = f_ref[pl.ds(i * blk_r, blk_r), :]
    nb = jnp.sqrt(jnp.sum(fb * fb, axis=1, keepdims=True))
    fbn = fb / jnp.maximum(nb, 1e-12)

    ft = ft_ref[:, :]
    nt = jnp.sqrt(jnp.sum(ft * ft, axis=0, keepdims=True))
    ftn = ft / jnp.maximum(nt, 1e-12)

    g = jax.lax.dot_general(
        fbn, ftn, (((1,), (0,)), ((), ())),
        preferred_element_type=jnp.float32,
    )
    sim = jnp.clip(g, -10.0, 10.0) * (1.0 / _TEMPERATURE)

    lrow = lrow_ref[:, :]
    lcol = lcol_ref[:, :]
    eq = lrow == lcol

    m = jnp.clip(jnp.sum(fbn * fbn, axis=1, keepdims=True),
                 -10.0, 10.0) * (1.0 / _TEMPERATURE)
    e = jnp.exp(sim - m)
    s_all = jnp.sum(e, axis=1, keepdims=True)
    ridx = jax.lax.broadcasted_iota(jnp.int32, (blk_r, 1), 0) + i * blk_r
    posf = jnp.where(eq & (cidx_ref[:, :] != ridx), 1.0, 0.0)
    exp_pos = jnp.sum(e * posf, axis=1, keepdims=True)
    cnt_same = jnp.sum(posf, axis=1, keepdims=True) + 1.0
    num_neg = jnp.float32(batch) - cnt_same
    s_same = exp_pos + 1.0
    s_allneg = (s_all - s_same) + cnt_same * jnp.exp(-m)

    n_chunks = batch // _LANES
    cand_rows = []
    for sb in range(blk_r // _SUB_R):
        r0 = sb * _SUB_R
        m1 = jnp.full((_SUB_R, _LANES), _NEG_INF)
        m2 = m1
        m3 = m1
        m4 = m1
        m5 = m1
        for c in range(n_chunks):
            c0 = c * _LANES
            v = jnp.where(eq[r0:r0 + _SUB_R, c0:c0 + _LANES], 0.0,
                          sim[r0:r0 + _SUB_R, c0:c0 + _LANES])
            t = jnp.maximum(m1, v)
            v = jnp.minimum(m1, v)
            m1 = t
            t = jnp.maximum(m2, v)
            v = jnp.minimum(m2, v)
            m2 = t
            t = jnp.maximum(m3, v)
            v = jnp.minimum(m3, v)
            m3 = t
            t = jnp.maximum(m4, v)
            v = jnp.minimum(m4, v)
            m4 = t
            m5 = jnp.maximum(m5, v)
        cand_rows.append(jnp.concatenate([m1, m2, m3, m4, m5], axis=1))
    cand = jnp.concatenate(cand_rows, axis=0)

    vals = []
    cnts = []
    work = cand
    for t in range(_NUM_HARD):
        mt = jnp.max(work, axis=1, keepdims=True)
        vals.append(mt)
        if t == _NUM_HARD - 1:
            break
        hit = work == mt
        ct = jnp.sum(jnp.where(hit, 1.0, 0.0), axis=1, keepdims=True)
        work = jnp.where(hit, _NEG_INF, work)
        cnts.append(ct)

    los = [jnp.zeros_like(m)]
    for t in range(1, _NUM_HARD):
        los.append(los[t - 1] + cnts[t - 1])
    s_hard = jnp.zeros_like(m)
    e_hard = []
    slot_vals = []
    for s in range(_NUM_HARD):
        sv = jnp.zeros_like(m)
        for t in range(_NUM_HARD):
            lo = los[t]
            if t == _NUM_HARD - 1:
                take = lo <= jnp.float32(s)
            else:
                take = (lo <= jnp.float32(s)) & (jnp.float32(s) < lo + cnts[t])
            sv = jnp.where(take, vals[t], sv)
        sv = jnp.where(jnp.float32(s) < jnp.minimum(num_neg, jnp.float32(_NUM_HARD)),
                       sv, 0.0)
        slot_vals.append(sv)

    for s in range(_NUM_HARD):
        eh = jnp.exp(slot_vals[s] - m)
        e_hard.append(eh)
        s_hard = s_hard + eh

    loss_blk = jnp.zeros((), jnp.float32)
    for s in range(_NUM_HARD):
        denom = exp_pos + s_hard + s_allneg - e_hard[s]
        lp = jnp.log(exp_pos / (denom + _EPS) + _EPS)
        loss_blk = loss_blk + jnp.sum(lp)

    @pl.when(i == 0)
    def _():
        out_ref[:, :] = jnp.zeros((1, 1), jnp.float32)

    out_ref[:, :] += loss_blk[None, None]


@jax.jit
def kernel(features, labels):
    batch, dim = features.shape
    blk_r = 256
    grid = batch // blk_r

    ft = features.T
    lrow = labels.reshape(batch, 1)
    lcol = labels.reshape(1, batch)
    cidx = jnp.arange(batch, dtype=jnp.int32).reshape(1, batch)

    body = functools.partial(_supcon_block_kernel, blk_r=blk_r, batch=batch)
    total = pl.pallas_call(
        body,
        grid=(grid,),
        in_specs=[
            pl.BlockSpec((batch, dim), lambda i: (0, 0)),
            pl.BlockSpec((dim, batch), lambda i: (0, 0)),
            pl.BlockSpec((blk_r, 1), lambda i: (i, 0)),
            pl.BlockSpec((1, batch), lambda i: (0, 0)),
            pl.BlockSpec((1, batch), lambda i: (0, 0)),
        ],
        out_specs=pl.BlockSpec((1, 1), lambda i: (0, 0)),
        out_shape=jax.ShapeDtypeStruct((1, 1), jnp.float32),
    )(features, ft, lrow, lcol, cidx)

    return -total[0, 0] / jnp.float32(batch * _NUM_HARD)


# speedup vs baseline: 4.9030x; 1.0237x over previous
import functools

import jax
import jax.numpy as jnp
from jax.experimental import pallas as pl
from jax.experimental.pallas import tpu as pltpu

_TEMPERATURE = 0.07
_NUM_HARD = 5
_EPS = 1e-08
_NEG_INF = -3.0e38
_LANES = 128
_SUB_R = 32


def _supcon_block_kernel(f_ref, ft_ref, lrow_ref, lcol_ref, cidx_ref, out_ref,
                         *, blk_r, batch):
    i = pl.program_id(0)

    fb = f_ref[pl.ds(i * blk_r, blk_r), :]
    nb = jnp.sqrt(jnp.sum(fb * fb, axis=1, keepdims=True))
    fbn = fb / jnp.maximum(nb, 1e-12)

    ft = ft_ref[:, :]
    nt = jnp.sqrt(jnp.sum(ft * ft, axis=0, keepdims=True))
    ftn = ft / jnp.maximum(nt, 1e-12)

    g = jax.lax.dot_general(
        fbn, ftn, (((1,), (0,)), ((), ())),
        preferred_element_type=jnp.float32,
    )
    sim = g * (1.0 / _TEMPERATURE)

    lrow = lrow_ref[:, :]
    lcol = lcol_ref[:, :]
    eq = lrow == lcol

    m = jnp.sum(fbn * fbn, axis=1, keepdims=True) * (1.0 / _TEMPERATURE)
    e = jnp.exp(sim - m)
    s_all = jnp.sum(e, axis=1, keepdims=True)
    ridx = jax.lax.broadcasted_iota(jnp.int32, (blk_r, 1), 0) + i * blk_r
    posf = jnp.where(eq & (cidx_ref[:, :] != ridx), 1.0, 0.0)
    exp_pos = jnp.sum(e * posf, axis=1, keepdims=True)
    cnt_same = jnp.sum(posf, axis=1, keepdims=True) + 1.0
    num_neg = jnp.float32(batch) - cnt_same
    s_same = exp_pos + 1.0
    s_allneg = (s_all - s_same) + cnt_same * jnp.exp(-m)

    n_chunks = batch // _LANES
    cand_rows = []
    for sb in range(blk_r // _SUB_R):
        r0 = sb * _SUB_R
        m1 = jnp.full((_SUB_R, _LANES), _NEG_INF)
        m2 = m1
        m3 = m1
        m4 = m1
        m5 = m1
        for c in range(n_chunks):
            c0 = c * _LANES
            v = jnp.where(eq[r0:r0 + _SUB_R, c0:c0 + _LANES], 0.0,
                          sim[r0:r0 + _SUB_R, c0:c0 + _LANES])
            t = jnp.maximum(m1, v)
            v = jnp.minimum(m1, v)
            m1 = t
            t = jnp.maximum(m2, v)
            v = jnp.minimum(m2, v)
            m2 = t
            t = jnp.maximum(m3, v)
            v = jnp.minimum(m3, v)
            m3 = t
            t = jnp.maximum(m4, v)
            v = jnp.minimum(m4, v)
            m4 = t
            m5 = jnp.maximum(m5, v)
        cand_rows.append(jnp.concatenate([m1, m2, m3, m4, m5], axis=1))
    cand = jnp.concatenate(cand_rows, axis=0)

    vals = []
    cnts = []
    work = cand
    for t in range(_NUM_HARD):
        mt = jnp.max(work, axis=1, keepdims=True)
        vals.append(mt)
        if t == _NUM_HARD - 1:
            break
        hit = work == mt
        ct = jnp.sum(jnp.where(hit, 1.0, 0.0), axis=1, keepdims=True)
        work = jnp.where(hit, _NEG_INF, work)
        cnts.append(ct)

    los = [jnp.zeros_like(m)]
    for t in range(1, _NUM_HARD):
        los.append(los[t - 1] + cnts[t - 1])
    s_hard = jnp.zeros_like(m)
    e_hard = []
    slot_vals = []
    for s in range(_NUM_HARD):
        sv = jnp.zeros_like(m)
        for t in range(_NUM_HARD):
            lo = los[t]
            if t == _NUM_HARD - 1:
                take = lo <= jnp.float32(s)
            else:
                take = (lo <= jnp.float32(s)) & (jnp.float32(s) < lo + cnts[t])
            sv = jnp.where(take, vals[t], sv)
        sv = jnp.where(jnp.float32(s) < jnp.minimum(num_neg, jnp.float32(_NUM_HARD)),
                       sv, 0.0)
        slot_vals.append(sv)

    for s in range(_NUM_HARD):
        eh = jnp.exp(slot_vals[s] - m)
        e_hard.append(eh)
        s_hard = s_hard + eh

    loss_blk = jnp.zeros((), jnp.float32)
    for s in range(_NUM_HARD):
        denom = exp_pos + s_hard + s_allneg - e_hard[s]
        lp = jnp.log(exp_pos / (denom + _EPS) + _EPS)
        loss_blk = loss_blk + jnp.sum(lp)

    @pl.when(i == 0)
    def _():
        out_ref[:, :] = jnp.zeros((1, 1), jnp.float32)

    out_ref[:, :] += loss_blk[None, None]


@jax.jit
def kernel(features, labels):
    batch, dim = features.shape
    blk_r = 256
    grid = batch // blk_r

    ft = features.T
    lrow = labels.reshape(batch, 1)
    lcol = labels.reshape(1, batch)
    cidx = jnp.arange(batch, dtype=jnp.int32).reshape(1, batch)

    body = functools.partial(_supcon_block_kernel, blk_r=blk_r, batch=batch)
    total = pl.pallas_call(
        body,
        grid=(grid,),
        in_specs=[
            pl.BlockSpec((batch, dim), lambda i: (0, 0)),
            pl.BlockSpec((dim, batch), lambda i: (0, 0)),
            pl.BlockSpec((blk_r, 1), lambda i: (i, 0)),
            pl.BlockSpec((1, batch), lambda i: (0, 0)),
            pl.BlockSpec((1, batch), lambda i: (0, 0)),
        ],
        out_specs=pl.BlockSpec((1, 1), lambda i: (0, 0)),
        out_shape=jax.ShapeDtypeStruct((1, 1), jnp.float32),
    )(features, ft, lrow, lcol, cidx)

    return -total[0, 0] / jnp.float32(batch * _NUM_HARD)


# speedup vs baseline: 5.1140x; 1.0430x over previous
import functools

import jax
import jax.numpy as jnp
from jax.experimental import pallas as pl
from jax.experimental.pallas import tpu as pltpu

_TEMPERATURE = 0.07
_NUM_HARD = 5
_EPS = 1e-08
_NEG_INF = -3.0e38
_LANES = 128
_SUB_R = 32


def _supcon_block_kernel(f_ref, ft_ref, lrow_ref, lcol_ref, cidx_ref, out_ref,
                         *, blk_r, batch):
    i = pl.program_id(0)

    fb = f_ref[pl.ds(i * blk_r, blk_r), :]
    nb = jnp.sqrt(jnp.sum(fb * fb, axis=1, keepdims=True))
    fbn = fb / jnp.maximum(nb, 1e-12)

    ft = ft_ref[:, :]
    nt = jnp.sqrt(jnp.sum(ft * ft, axis=0, keepdims=True))
    ftn = ft / jnp.maximum(nt, 1e-12)

    g = jax.lax.dot_general(
        fbn, ftn, (((1,), (0,)), ((), ())),
        preferred_element_type=jnp.float32,
    )
    sim = g * (1.0 / _TEMPERATURE)

    lrow = lrow_ref[:, :]
    lcol = lcol_ref[:, :]
    eq = lrow == lcol

    m = jnp.sum(fbn * fbn, axis=1, keepdims=True) * (1.0 / _TEMPERATURE)
    e = jnp.exp(sim - m)
    s_all = jnp.sum(e, axis=1, keepdims=True)
    ridx = jax.lax.broadcasted_iota(jnp.int32, (blk_r, 1), 0) + i * blk_r
    posf = jnp.where(eq & (cidx_ref[:, :] != ridx), 1.0, 0.0)
    exp_pos = jnp.sum(e * posf, axis=1, keepdims=True)
    cnt_same = jnp.sum(posf, axis=1, keepdims=True) + 1.0
    num_neg = jnp.float32(batch) - cnt_same
    s_same = exp_pos + 1.0
    s_allneg = (s_all - s_same) + cnt_same * jnp.exp(-m)

    n_chunks = batch // _LANES
    cand_rows = []
    for sb in range(blk_r // _SUB_R):
        r0 = sb * _SUB_R
        ms = [jnp.full((_SUB_R, _LANES), _NEG_INF)] * _NUM_HARD
        for gb in range(n_chunks // 8):
            vs = []
            for k in range(8):
                c0 = (gb * 8 + k) * _LANES
                vs.append(jnp.where(eq[r0:r0 + _SUB_R, c0:c0 + _LANES], 0.0,
                                    sim[r0:r0 + _SUB_R, c0:c0 + _LANES]))
            p1 = jnp.maximum(vs[0], vs[1])
            p2 = jnp.minimum(vs[0], vs[1])
            q1 = jnp.maximum(vs[2], vs[3])
            q2 = jnp.minimum(vs[2], vs[3])
            r1 = jnp.maximum(vs[4], vs[5])
            r2 = jnp.minimum(vs[4], vs[5])
            s1 = jnp.maximum(vs[6], vs[7])
            s2 = jnp.minimum(vs[6], vs[7])
            a1 = jnp.maximum(p1, q1)
            t1 = jnp.minimum(p1, q1)
            a4 = jnp.minimum(p2, q2)
            t2 = jnp.maximum(p2, q2)
            a2 = jnp.maximum(t1, t2)
            a3 = jnp.minimum(t1, t2)
            b1 = jnp.maximum(r1, s1)
            t1 = jnp.minimum(r1, s1)
            b4 = jnp.minimum(r2, s2)
            t2 = jnp.maximum(r2, s2)
            b2 = jnp.maximum(t1, t2)
            b3 = jnp.minimum(t1, t2)
            o1 = jnp.maximum(a1, b1)
            t1 = jnp.minimum(a1, b1)
            t2 = jnp.maximum(a3, b3)
            o2 = jnp.maximum(t1, t2)
            o3 = jnp.minimum(t1, t2)
            e1 = jnp.maximum(a2, b2)
            u1 = jnp.minimum(a2, b2)
            u2 = jnp.maximum(a4, b4)
            e2 = jnp.maximum(u1, u2)
            c1 = o1
            c2 = jnp.maximum(e1, o2)
            c3 = jnp.minimum(e1, o2)
            c4 = jnp.maximum(e2, o3)
            c5 = jnp.minimum(e2, o3)
            for k, x in enumerate((c1, c2, c3, c4, c5)):
                for j in range(k, _NUM_HARD - 1):
                    hi = jnp.maximum(ms[j], x)
                    x = jnp.minimum(ms[j], x)
                    ms[j] = hi
                ms[_NUM_HARD - 1] = jnp.maximum(ms[_NUM_HARD - 1], x)
        cand_rows.append(jnp.concatenate(ms, axis=1))
    cand = jnp.concatenate(cand_rows, axis=0)

    vals = []
    cnts = []
    work = cand
    for t in range(_NUM_HARD):
        mt = jnp.max(work, axis=1, keepdims=True)
        vals.append(mt)
        if t == _NUM_HARD - 1:
            break
        hit = work == mt
        ct = jnp.sum(jnp.where(hit, 1.0, 0.0), axis=1, keepdims=True)
        work = jnp.where(hit, _NEG_INF, work)
        cnts.append(ct)

    los = [jnp.zeros_like(m)]
    for t in range(1, _NUM_HARD):
        los.append(los[t - 1] + cnts[t - 1])
    s_hard = jnp.zeros_like(m)
    e_hard = []
    slot_vals = []
    for s in range(_NUM_HARD):
        sv = jnp.zeros_like(m)
        for t in range(_NUM_HARD):
            lo = los[t]
            if t == _NUM_HARD - 1:
                take = lo <= jnp.float32(s)
            else:
                take = (lo <= jnp.float32(s)) & (jnp.float32(s) < lo + cnts[t])
            sv = jnp.where(take, vals[t], sv)
        sv = jnp.where(jnp.float32(s) < jnp.minimum(num_neg, jnp.float32(_NUM_HARD)),
                       sv, 0.0)
        slot_vals.append(sv)

    for s in range(_NUM_HARD):
        eh = jnp.exp(slot_vals[s] - m)
        e_hard.append(eh)
        s_hard = s_hard + eh

    loss_blk = jnp.zeros((), jnp.float32)
    for s in range(_NUM_HARD):
        denom = exp_pos + s_hard + s_allneg - e_hard[s]
        lp = jnp.log(exp_pos / (denom + _EPS) + _EPS)
        loss_blk = loss_blk + jnp.sum(lp)

    @pl.when(i == 0)
    def _():
        out_ref[:, :] = jnp.zeros((1, 1), jnp.float32)

    out_ref[:, :] += loss_blk[None, None]


@jax.jit
def kernel(features, labels):
    batch, dim = features.shape
    blk_r = 256
    grid = batch // blk_r

    ft = features.T
    lrow = labels.reshape(batch, 1)
    lcol = labels.reshape(1, batch)
    cidx = jnp.arange(batch, dtype=jnp.int32).reshape(1, batch)

    body = functools.partial(_supcon_block_kernel, blk_r=blk_r, batch=batch)
    total = pl.pallas_call(
        body,
        grid=(grid,),
        in_specs=[
            pl.BlockSpec((batch, dim), lambda i: (0, 0)),
            pl.BlockSpec((dim, batch), lambda i: (0, 0)),
            pl.BlockSpec((blk_r, 1), lambda i: (i, 0)),
            pl.BlockSpec((1, batch), lambda i: (0, 0)),
            pl.BlockSpec((1, batch), lambda i: (0, 0)),
        ],
        out_specs=pl.BlockSpec((1, 1), lambda i: (0, 0)),
        out_shape=jax.ShapeDtypeStruct((1, 1), jnp.float32),
    )(features, ft, lrow, lcol, cidx)

    return -total[0, 0] / jnp.float32(batch * _NUM_HARD)


# speedup vs baseline: 5.2395x; 1.0245x over previous
import functools

import jax
import jax.numpy as jnp
from jax.experimental import pallas as pl
from jax.experimental.pallas import tpu as pltpu

_TEMPERATURE = 0.07
_NUM_HARD = 5
_EPS = 1e-08
_NEG_INF = -3.0e38
_LANES = 128
_SUB_R = 32


def _supcon_block_kernel(f_ref, ft_ref, lrow_ref, lcol_ref, cidx_ref, out_ref,
                         *, blk_r, batch):
    i = pl.program_id(0)

    fb = f_ref[pl.ds(i * blk_r, blk_r), :]
    nb = jnp.sqrt(jnp.sum(fb * fb, axis=1, keepdims=True))
    fbn = fb / jnp.maximum(nb, 1e-12)

    ft = ft_ref[:, :]
    nt = jnp.sqrt(jnp.sum(ft * ft, axis=0, keepdims=True))
    ftn = ft / jnp.maximum(nt, 1e-12)

    sim = jax.lax.dot_general(
        fbn * (1.0 / _TEMPERATURE), ftn, (((1,), (0,)), ((), ())),
        preferred_element_type=jnp.float32,
    )

    lrow = lrow_ref[:, :]
    lcol = lcol_ref[:, :]
    eq = lrow == lcol

    m = jnp.sum(fbn * fbn, axis=1, keepdims=True) * (1.0 / _TEMPERATURE)
    e = jnp.exp(sim - m)
    s_all = jnp.sum(e, axis=1, keepdims=True)
    ridx = jax.lax.broadcasted_iota(jnp.int32, (blk_r, 1), 0) + i * blk_r
    posf = jnp.where(eq & (cidx_ref[:, :] != ridx), 1.0, 0.0)
    exp_pos = jnp.sum(e * posf, axis=1, keepdims=True)
    cnt_same = jnp.sum(posf, axis=1, keepdims=True) + 1.0
    num_neg = jnp.float32(batch) - cnt_same
    s_same = exp_pos + 1.0
    s_allneg = (s_all - s_same) + cnt_same * jnp.exp(-m)

    n_chunks = batch // _LANES
    cand_rows = []
    for sb in range(blk_r // _SUB_R):
        r0 = sb * _SUB_R
        ms = [jnp.full((_SUB_R, _LANES), _NEG_INF)] * _NUM_HARD
        for gb in range(n_chunks // 8):
            vs = []
            for k in range(8):
                c0 = (gb * 8 + k) * _LANES
                vs.append(jnp.where(eq[r0:r0 + _SUB_R, c0:c0 + _LANES], 0.0,
                                    sim[r0:r0 + _SUB_R, c0:c0 + _LANES]))
            p1 = jnp.maximum(vs[0], vs[1])
            p2 = jnp.minimum(vs[0], vs[1])
            q1 = jnp.maximum(vs[2], vs[3])
            q2 = jnp.minimum(vs[2], vs[3])
            r1 = jnp.maximum(vs[4], vs[5])
            r2 = jnp.minimum(vs[4], vs[5])
            s1 = jnp.maximum(vs[6], vs[7])
            s2 = jnp.minimum(vs[6], vs[7])
            a1 = jnp.maximum(p1, q1)
            t1 = jnp.minimum(p1, q1)
            a4 = jnp.minimum(p2, q2)
            t2 = jnp.maximum(p2, q2)
            a2 = jnp.maximum(t1, t2)
            a3 = jnp.minimum(t1, t2)
            b1 = jnp.maximum(r1, s1)
            t1 = jnp.minimum(r1, s1)
            b4 = jnp.minimum(r2, s2)
            t2 = jnp.maximum(r2, s2)
            b2 = jnp.maximum(t1, t2)
            b3 = jnp.minimum(t1, t2)
            o1 = jnp.maximum(a1, b1)
            t1 = jnp.minimum(a1, b1)
            t2 = jnp.maximum(a3, b3)
            o2 = jnp.maximum(t1, t2)
            o3 = jnp.minimum(t1, t2)
            e1 = jnp.maximum(a2, b2)
            u1 = jnp.minimum(a2, b2)
            u2 = jnp.maximum(a4, b4)
            e2 = jnp.maximum(u1, u2)
            c1 = o1
            c2 = jnp.maximum(e1, o2)
            c3 = jnp.minimum(e1, o2)
            c4 = jnp.maximum(e2, o3)
            c5 = jnp.minimum(e2, o3)
            for k, x in enumerate((c1, c2, c3, c4, c5)):
                for j in range(k, _NUM_HARD - 1):
                    hi = jnp.maximum(ms[j], x)
                    x = jnp.minimum(ms[j], x)
                    ms[j] = hi
                ms[_NUM_HARD - 1] = jnp.maximum(ms[_NUM_HARD - 1], x)
        cand_rows.append(jnp.concatenate(ms, axis=1))
    cand = jnp.concatenate(cand_rows, axis=0)

    vals = []
    cnts = []
    work = cand
    for t in range(_NUM_HARD):
        mt = jnp.max(work, axis=1, keepdims=True)
        vals.append(mt)
        if t == _NUM_HARD - 1:
            break
        hit = work == mt
        ct = jnp.sum(jnp.where(hit, 1.0, 0.0), axis=1, keepdims=True)
        work = jnp.where(hit, _NEG_INF, work)
        cnts.append(ct)

    los = [jnp.zeros_like(m)]
    for t in range(1, _NUM_HARD):
        los.append(los[t - 1] + cnts[t - 1])
    s_hard = jnp.zeros_like(m)
    e_hard = []
    slot_vals = []
    for s in range(_NUM_HARD):
        sv = jnp.zeros_like(m)
        for t in range(_NUM_HARD):
            lo = los[t]
            if t == _NUM_HARD - 1:
                take = lo <= jnp.float32(s)
            else:
                take = (lo <= jnp.float32(s)) & (jnp.float32(s) < lo + cnts[t])
            sv = jnp.where(take, vals[t], sv)
        sv = jnp.where(jnp.float32(s) < jnp.minimum(num_neg, jnp.float32(_NUM_HARD)),
                       sv, 0.0)
        slot_vals.append(sv)

    for s in range(_NUM_HARD):
        eh = jnp.exp(slot_vals[s] - m)
        e_hard.append(eh)
        s_hard = s_hard + eh

    loss_blk = jnp.zeros((), jnp.float32)
    for s in range(_NUM_HARD):
        denom = exp_pos + s_hard + s_allneg - e_hard[s]
        lp = jnp.log(exp_pos / (denom + _EPS) + _EPS)
        loss_blk = loss_blk + jnp.sum(lp)

    @pl.when(i == 0)
    def _():
        out_ref[:, :] = jnp.zeros((1, 1), jnp.float32)

    out_ref[:, :] += loss_blk[None, None]


@jax.jit
def kernel(features, labels):
    batch, dim = features.shape
    blk_r = 256
    grid = batch // blk_r

    ft = features.T
    lrow = labels.reshape(batch, 1)
    lcol = labels.reshape(1, batch)
    cidx = jnp.arange(batch, dtype=jnp.int32).reshape(1, batch)

    body = functools.partial(_supcon_block_kernel, blk_r=blk_r, batch=batch)
    total = pl.pallas_call(
        body,
        grid=(grid,),
        in_specs=[
            pl.BlockSpec((batch, dim), lambda i: (0, 0)),
            pl.BlockSpec((dim, batch), lambda i: (0, 0)),
            pl.BlockSpec((blk_r, 1), lambda i: (i, 0)),
            pl.BlockSpec((1, batch), lambda i: (0, 0)),
            pl.BlockSpec((1, batch), lambda i: (0, 0)),
        ],
        out_specs=pl.BlockSpec((1, 1), lambda i: (0, 0)),
        out_shape=jax.ShapeDtypeStruct((1, 1), jnp.float32),
    )(features, ft, lrow, lcol, cidx)

    return -total[0, 0] / jnp.float32(batch * _NUM_HARD)


# speedup vs baseline: 5.4296x; 1.0363x over previous
import functools

import jax
import jax.numpy as jnp
from jax.experimental import pallas as pl

_TEMPERATURE = 0.07
_NUM_HARD = 5
_EPS = 1e-08
_NEG_INF = -3.0e38
_LANES = 128
_SUB_R = 32


def _supcon_block_kernel(f_ref, ft_ref, lrow_ref, lcol_ref, cidx_ref, out_ref,
                         *, blk_r, batch):
    i = pl.program_id(0)

    fb = f_ref[pl.ds(i * blk_r, blk_r), :]
    nb = jnp.sqrt(jnp.sum(fb * fb, axis=1, keepdims=True))
    fbn = fb / jnp.maximum(nb, 1e-12)

    ft = ft_ref[:, :]
    nt = jnp.sqrt(jnp.sum(ft * ft, axis=0, keepdims=True))
    ftn = ft / jnp.maximum(nt, 1e-12)

    sim = jax.lax.dot_general(
        fbn * (1.0 / _TEMPERATURE), ftn, (((1,), (0,)), ((), ())),
        preferred_element_type=jnp.float32,
    )

    lrow = lrow_ref[:, :]
    lcol = lcol_ref[:, :]
    eq = lrow == lcol

    m = jnp.sum(fbn * fbn, axis=1, keepdims=True) * (1.0 / _TEMPERATURE)
    e = jnp.exp(sim - m)
    s_all = jnp.sum(e, axis=1, keepdims=True)
    ridx = jax.lax.broadcasted_iota(jnp.int32, (blk_r, 1), 0) + i * blk_r
    posf = jnp.where(eq & (cidx_ref[:, :] != ridx), 1.0, 0.0)
    exp_pos = jnp.sum(e * posf, axis=1, keepdims=True)
    cnt_same = jnp.sum(posf, axis=1, keepdims=True) + 1.0
    num_neg = jnp.float32(batch) - cnt_same
    s_same = exp_pos + 1.0
    s_allneg = (s_all - s_same) + cnt_same * jnp.exp(-m)

    n_chunks = batch // _LANES
    cand_rows = []
    for sb in range(blk_r // _SUB_R):
        r0 = sb * _SUB_R
        ms = [jnp.full((_SUB_R, _LANES), _NEG_INF)] * _NUM_HARD
        for gb in range(n_chunks // 8):
            vs = []
            for k in range(8):
                c0 = (gb * 8 + k) * _LANES
                vs.append(jnp.where(eq[r0:r0 + _SUB_R, c0:c0 + _LANES], 0.0,
                                    sim[r0:r0 + _SUB_R, c0:c0 + _LANES]))
            p1 = jnp.maximum(vs[0], vs[1])
            p2 = jnp.minimum(vs[0], vs[1])
            q1 = jnp.maximum(vs[2], vs[3])
            q2 = jnp.minimum(vs[2], vs[3])
            r1 = jnp.maximum(vs[4], vs[5])
            r2 = jnp.minimum(vs[4], vs[5])
            s1 = jnp.maximum(vs[6], vs[7])
            s2 = jnp.minimum(vs[6], vs[7])
            a1 = jnp.maximum(p1, q1)
            t1 = jnp.minimum(p1, q1)
            a4 = jnp.minimum(p2, q2)
            t2 = jnp.maximum(p2, q2)
            a2 = jnp.maximum(t1, t2)
            a3 = jnp.minimum(t1, t2)
            b1 = jnp.maximum(r1, s1)
            t1 = jnp.minimum(r1, s1)
            b4 = jnp.minimum(r2, s2)
            t2 = jnp.maximum(r2, s2)
            b2 = jnp.maximum(t1, t2)
            b3 = jnp.minimum(t1, t2)
            o1 = jnp.maximum(a1, b1)
            t1 = jnp.minimum(a1, b1)
            t2 = jnp.maximum(a3, b3)
            o2 = jnp.maximum(t1, t2)
            o3 = jnp.minimum(t1, t2)
            e1 = jnp.maximum(a2, b2)
            u1 = jnp.minimum(a2, b2)
            u2 = jnp.maximum(a4, b4)
            e2 = jnp.maximum(u1, u2)
            c1 = o1
            c2 = jnp.maximum(e1, o2)
            c3 = jnp.minimum(e1, o2)
            c4 = jnp.maximum(e2, o3)
            c5 = jnp.minimum(e2, o3)
            for k, x in enumerate((c1, c2, c3, c4, c5)):
                for j in range(k, _NUM_HARD - 1):
                    hi = jnp.maximum(ms[j], x)
                    x = jnp.minimum(ms[j], x)
                    ms[j] = hi
                ms[_NUM_HARD - 1] = jnp.maximum(ms[_NUM_HARD - 1], x)
        cand_rows.append(jnp.concatenate(ms, axis=1))
    cand = jnp.concatenate(cand_rows, axis=0)

    vals = []
    cnts = []
    work = cand
    for t in range(_NUM_HARD):
        mt = jnp.max(work, axis=1, keepdims=True)
        vals.append(mt)
        if t == _NUM_HARD - 1:
            break
        hit = work == mt
        ct = jnp.sum(jnp.where(hit, 1.0, 0.0), axis=1, keepdims=True)
        work = jnp.where(hit, _NEG_INF, work)
        cnts.append(ct)

    los = [jnp.zeros_like(m)]
    for t in range(1, _NUM_HARD):
        los.append(los[t - 1] + cnts[t - 1])
    s_hard = jnp.zeros_like(m)
    e_hard = []
    slot_vals = []
    for s in range(_NUM_HARD):
        sv = jnp.zeros_like(m)
        for t in range(_NUM_HARD):
            lo = los[t]
            if t == _NUM_HARD - 1:
                take = lo <= jnp.float32(s)
            else:
                take = (lo <= jnp.float32(s)) & (jnp.float32(s) < lo + cnts[t])
            sv = jnp.where(take, vals[t], sv)
        sv = jnp.where(jnp.float32(s) < jnp.minimum(num_neg, jnp.float32(_NUM_HARD)),
                       sv, 0.0)
        slot_vals.append(sv)

    for s in range(_NUM_HARD):
        eh = jnp.exp(slot_vals[s] - m)
        e_hard.append(eh)
        s_hard = s_hard + eh

    loss_blk = jnp.zeros((), jnp.float32)
    for s in range(_NUM_HARD):
        denom = exp_pos + s_hard + s_allneg - e_hard[s]
        lp = jnp.log(exp_pos / (denom + _EPS) + _EPS)
        loss_blk = loss_blk + jnp.sum(lp)

    @pl.when(i == 0)
    def _():
        out_ref[:, :] = jnp.zeros((1, 1), jnp.float32)

    out_ref[:, :] += loss_blk[None, None]


@jax.jit
def kernel(features, labels):
    batch, dim = features.shape
    blk_r = 512
    grid = batch // blk_r

    ft = features.T
    lrow = labels.reshape(batch, 1)
    lcol = labels.reshape(1, batch)
    cidx = jnp.arange(batch, dtype=jnp.int32).reshape(1, batch)

    body = functools.partial(_supcon_block_kernel, blk_r=blk_r, batch=batch)
    total = pl.pallas_call(
        body,
        grid=(grid,),
        in_specs=[
            pl.BlockSpec((batch, dim), lambda i: (0, 0)),
            pl.BlockSpec((dim, batch), lambda i: (0, 0)),
            pl.BlockSpec((blk_r, 1), lambda i: (i, 0)),
            pl.BlockSpec((1, batch), lambda i: (0, 0)),
            pl.BlockSpec((1, batch), lambda i: (0, 0)),
        ],
        out_specs=pl.BlockSpec((1, 1), lambda i: (0, 0)),
        out_shape=jax.ShapeDtypeStruct((1, 1), jnp.float32),
    )(features, ft, lrow, lcol, cidx)

    return -total[0, 0] / jnp.float32(batch * _NUM_HARD)


# speedup vs baseline: 5.5496x; 1.0221x over previous
import functools

import jax
import jax.numpy as jnp
from jax.experimental import pallas as pl

_TEMPERATURE = 0.07
_NUM_HARD = 5
_EPS = 1e-08
_NEG_INF = -3.0e38
_LANES = 128
_SUB_R = 32


def _supcon_block_kernel(f_ref, ft_ref, lrow_ref, lcol_ref, cidx_ref, out_ref,
                         *, blk_r, batch):
    i = pl.program_id(0)

    fb = f_ref[pl.ds(i * blk_r, blk_r), :]
    nb = jnp.sqrt(jnp.sum(fb * fb, axis=1, keepdims=True))
    fbn = fb / jnp.maximum(nb, 1e-12)

    ft = ft_ref[:, :]
    nt = jnp.sqrt(jnp.sum(ft * ft, axis=0, keepdims=True))
    ftn = ft / jnp.maximum(nt, 1e-12)

    sim = jax.lax.dot_general(
        fbn * (1.0 / _TEMPERATURE), ftn, (((1,), (0,)), ((), ())),
        preferred_element_type=jnp.float32,
    )

    lrow = lrow_ref[:, :]
    lcol = lcol_ref[:, :]
    eq = lrow == lcol

    m = jnp.sum(fbn * fbn, axis=1, keepdims=True) * (1.0 / _TEMPERATURE)
    e = jnp.exp(sim - m)
    s_all = jnp.sum(e, axis=1, keepdims=True)
    ridx = jax.lax.broadcasted_iota(jnp.int32, (blk_r, 1), 0) + i * blk_r
    posf = jnp.where(eq & (cidx_ref[:, :] != ridx), 1.0, 0.0)
    exp_pos = jnp.sum(e * posf, axis=1, keepdims=True)
    cnt_same = jnp.sum(posf, axis=1, keepdims=True) + 1.0
    num_neg = jnp.float32(batch) - cnt_same
    s_same = exp_pos + 1.0
    s_allneg = (s_all - s_same) + cnt_same * jnp.exp(-m)

    n_chunks = batch // _LANES
    cand_rows = []
    for sb in range(blk_r // _SUB_R):
        r0 = sb * _SUB_R
        ms = [jnp.full((_SUB_R, _LANES), _NEG_INF)] * _NUM_HARD
        for gb in range(n_chunks // 8):
            vs = []
            for k in range(8):
                c0 = (gb * 8 + k) * _LANES
                vs.append(jnp.where(eq[r0:r0 + _SUB_R, c0:c0 + _LANES], 0.0,
                                    sim[r0:r0 + _SUB_R, c0:c0 + _LANES]))
            p1 = jnp.maximum(vs[0], vs[1])
            p2 = jnp.minimum(vs[0], vs[1])
            q1 = jnp.maximum(vs[2], vs[3])
            q2 = jnp.minimum(vs[2], vs[3])
            r1 = jnp.maximum(vs[4], vs[5])
            r2 = jnp.minimum(vs[4], vs[5])
            s1 = jnp.maximum(vs[6], vs[7])
            s2 = jnp.minimum(vs[6], vs[7])
            a1 = jnp.maximum(p1, q1)
            t1 = jnp.minimum(p1, q1)
            a4 = jnp.minimum(p2, q2)
            t2 = jnp.maximum(p2, q2)
            a2 = jnp.maximum(t1, t2)
            a3 = jnp.minimum(t1, t2)
            b1 = jnp.maximum(r1, s1)
            t1 = jnp.minimum(r1, s1)
            b4 = jnp.minimum(r2, s2)
            t2 = jnp.maximum(r2, s2)
            b2 = jnp.maximum(t1, t2)
            b3 = jnp.minimum(t1, t2)
            o1 = jnp.maximum(a1, b1)
            t1 = jnp.minimum(a1, b1)
            t2 = jnp.maximum(a3, b3)
            o2 = jnp.maximum(t1, t2)
            o3 = jnp.minimum(t1, t2)
            e1 = jnp.maximum(a2, b2)
            u1 = jnp.minimum(a2, b2)
            u2 = jnp.maximum(a4, b4)
            e2 = jnp.maximum(u1, u2)
            c1 = o1
            c2 = jnp.maximum(e1, o2)
            c3 = jnp.minimum(e1, o2)
            c4 = jnp.maximum(e2, o3)
            c5 = jnp.minimum(e2, o3)
            for k, x in enumerate((c1, c2, c3, c4, c5)):
                for j in range(k, _NUM_HARD - 1):
                    hi = jnp.maximum(ms[j], x)
                    x = jnp.minimum(ms[j], x)
                    ms[j] = hi
                ms[_NUM_HARD - 1] = jnp.maximum(ms[_NUM_HARD - 1], x)
        cand_rows.append(jnp.concatenate(ms, axis=1))
    cand = jnp.concatenate(cand_rows, axis=0)

    vals = []
    cnts = []
    work = cand
    for t in range(_NUM_HARD):
        mt = jnp.max(work, axis=1, keepdims=True)
        vals.append(mt)
        if t == _NUM_HARD - 1:
            break
        hit = work == mt
        ct = jnp.sum(jnp.where(hit, 1.0, 0.0), axis=1, keepdims=True)
        work = jnp.where(hit, _NEG_INF, work)
        cnts.append(ct)

    los = [jnp.zeros_like(m)]
    for t in range(1, _NUM_HARD):
        los.append(los[t - 1] + cnts[t - 1])
    s_hard = jnp.zeros_like(m)
    e_hard = []
    slot_vals = []
    for s in range(_NUM_HARD):
        sv = jnp.zeros_like(m)
        for t in range(_NUM_HARD):
            lo = los[t]
            if t == _NUM_HARD - 1:
                take = lo <= jnp.float32(s)
            else:
                take = (lo <= jnp.float32(s)) & (jnp.float32(s) < lo + cnts[t])
            sv = jnp.where(take, vals[t], sv)
        sv = jnp.where(jnp.float32(s) < jnp.minimum(num_neg, jnp.float32(_NUM_HARD)),
                       sv, 0.0)
        slot_vals.append(sv)

    for s in range(_NUM_HARD):
        eh = jnp.exp(slot_vals[s] - m)
        e_hard.append(eh)
        s_hard = s_hard + eh

    loss_blk = jnp.zeros((), jnp.float32)
    for s in range(_NUM_HARD):
        denom = exp_pos + s_hard + s_allneg - e_hard[s]
        lp = jnp.log(exp_pos / (denom + _EPS) + _EPS)
        loss_blk = loss_blk + jnp.sum(lp)

    @pl.when(i == 0)
    def _():
        out_ref[:, :] = jnp.zeros((1, 1), jnp.float32)

    out_ref[:, :] += loss_blk[None, None]


@jax.jit
def kernel(features, labels):
    batch, dim = features.shape
    blk_r = 1024
    grid = batch // blk_r

    ft = features.T
    lrow = labels.reshape(batch, 1)
    lcol = labels.reshape(1, batch)
    cidx = jnp.arange(batch, dtype=jnp.int32).reshape(1, batch)

    body = functools.partial(_supcon_block_kernel, blk_r=blk_r, batch=batch)
    total = pl.pallas_call(
        body,
        grid=(grid,),
        in_specs=[
            pl.BlockSpec((batch, dim), lambda i: (0, 0)),
            pl.BlockSpec((dim, batch), lambda i: (0, 0)),
            pl.BlockSpec((blk_r, 1), lambda i: (i, 0)),
            pl.BlockSpec((1, batch), lambda i: (0, 0)),
            pl.BlockSpec((1, batch), lambda i: (0, 0)),
        ],
        out_specs=pl.BlockSpec((1, 1), lambda i: (0, 0)),
        out_shape=jax.ShapeDtypeStruct((1, 1), jnp.float32),
    )(features, ft, lrow, lcol, cidx)

    return -total[0, 0] / jnp.float32(batch * _NUM_HARD)


# speedup vs baseline: 5.6858x; 1.0245x over previous
import functools

import jax
import jax.numpy as jnp
from jax.experimental import pallas as pl

_TEMPERATURE = 0.07
_NUM_HARD = 5
_EPS = 1e-08
_NEG_INF = -3.0e38
_LANES = 128
_SUB_R = 32


def _supcon_block_kernel(f_ref, ft_ref, lrow_ref, lcol_ref, cidx_ref, out_ref,
                         *, blk_r, batch):
    i = pl.program_id(0)

    fb = f_ref[pl.ds(i * blk_r, blk_r), :]
    nb = jnp.sqrt(jnp.sum(fb * fb, axis=1, keepdims=True))
    fbn = fb / jnp.maximum(nb, 1e-12)

    ft = ft_ref[:, :]
    nt = jnp.sqrt(jnp.sum(ft * ft, axis=0, keepdims=True))
    ftn = ft / jnp.maximum(nt, 1e-12)

    sim = jax.lax.dot_general(
        fbn * (1.0 / _TEMPERATURE), ftn, (((1,), (0,)), ((), ())),
        preferred_element_type=jnp.float32,
    )

    lrow = lrow_ref[:, :]
    lcol = lcol_ref[:, :]
    eq = lrow == lcol

    m = jnp.sum(fbn * fbn, axis=1, keepdims=True) * (1.0 / _TEMPERATURE)
    e = jnp.exp(sim - m)
    s_all = jnp.sum(e, axis=1, keepdims=True)
    ridx = jax.lax.broadcasted_iota(jnp.int32, (blk_r, 1), 0) + i * blk_r
    posf = jnp.where(eq & (cidx_ref[:, :] != ridx), 1.0, 0.0)
    exp_pos = jnp.sum(e * posf, axis=1, keepdims=True)
    cnt_same = jnp.sum(posf, axis=1, keepdims=True) + 1.0
    num_neg = jnp.float32(batch) - cnt_same
    s_same = exp_pos + 1.0
    s_allneg = (s_all - s_same) + cnt_same * jnp.exp(-m)

    n_chunks = batch // _LANES
    cand_rows = []
    for sb in range(blk_r // _SUB_R):
        r0 = sb * _SUB_R
        ms = [jnp.full((_SUB_R, _LANES), _NEG_INF)] * _NUM_HARD
        for gb in range(n_chunks // 8):
            vs = []
            for k in range(8):
                c0 = (gb * 8 + k) * _LANES
                vs.append(jnp.where(eq[r0:r0 + _SUB_R, c0:c0 + _LANES], 0.0,
                                    sim[r0:r0 + _SUB_R, c0:c0 + _LANES]))
            p1 = jnp.maximum(vs[0], vs[1])
            p2 = jnp.minimum(vs[0], vs[1])
            q1 = jnp.maximum(vs[2], vs[3])
            q2 = jnp.minimum(vs[2], vs[3])
            r1 = jnp.maximum(vs[4], vs[5])
            r2 = jnp.minimum(vs[4], vs[5])
            s1 = jnp.maximum(vs[6], vs[7])
            s2 = jnp.minimum(vs[6], vs[7])
            a1 = jnp.maximum(p1, q1)
            t1 = jnp.minimum(p1, q1)
            a4 = jnp.minimum(p2, q2)
            t2 = jnp.maximum(p2, q2)
            a2 = jnp.maximum(t1, t2)
            a3 = jnp.minimum(t1, t2)
            b1 = jnp.maximum(r1, s1)
            t1 = jnp.minimum(r1, s1)
            b4 = jnp.minimum(r2, s2)
            t2 = jnp.maximum(r2, s2)
            b2 = jnp.maximum(t1, t2)
            b3 = jnp.minimum(t1, t2)
            o1 = jnp.maximum(a1, b1)
            t1 = jnp.minimum(a1, b1)
            t2 = jnp.maximum(a3, b3)
            o2 = jnp.maximum(t1, t2)
            o3 = jnp.minimum(t1, t2)
            e1 = jnp.maximum(a2, b2)
            u1 = jnp.minimum(a2, b2)
            u2 = jnp.maximum(a4, b4)
            e2 = jnp.maximum(u1, u2)
            c1 = o1
            c2 = jnp.maximum(e1, o2)
            c3 = jnp.minimum(e1, o2)
            c4 = jnp.maximum(e2, o3)
            c5 = jnp.minimum(e2, o3)
            for k, x in enumerate((c1, c2, c3, c4, c5)):
                for j in range(k, _NUM_HARD - 1):
                    hi = jnp.maximum(ms[j], x)
                    x = jnp.minimum(ms[j], x)
                    ms[j] = hi
                ms[_NUM_HARD - 1] = jnp.maximum(ms[_NUM_HARD - 1], x)
        cand_rows.append(jnp.concatenate(ms, axis=1))
    cand = jnp.concatenate(cand_rows, axis=0)

    vals = []
    cnts = []
    work = cand
    for t in range(_NUM_HARD):
        mt = jnp.max(work, axis=1, keepdims=True)
        vals.append(mt)
        if t == _NUM_HARD - 1:
            break
        hit = work == mt
        ct = jnp.sum(jnp.where(hit, 1.0, 0.0), axis=1, keepdims=True)
        work = jnp.where(hit, _NEG_INF, work)
        cnts.append(ct)

    los = [jnp.zeros_like(m)]
    for t in range(1, _NUM_HARD):
        los.append(los[t - 1] + cnts[t - 1])
    s_hard = jnp.zeros_like(m)
    e_hard = []
    slot_vals = []
    for s in range(_NUM_HARD):
        sv = jnp.zeros_like(m)
        for t in range(_NUM_HARD):
            lo = los[t]
            if t == _NUM_HARD - 1:
                take = lo <= jnp.float32(s)
            else:
                take = (lo <= jnp.float32(s)) & (jnp.float32(s) < lo + cnts[t])
            sv = jnp.where(take, vals[t], sv)
        sv = jnp.where(jnp.float32(s) < jnp.minimum(num_neg, jnp.float32(_NUM_HARD)),
                       sv, 0.0)
        slot_vals.append(sv)

    for s in range(_NUM_HARD):
        eh = jnp.exp(slot_vals[s] - m)
        e_hard.append(eh)
        s_hard = s_hard + eh

    loss_blk = jnp.zeros((), jnp.float32)
    for s in range(_NUM_HARD):
        denom = exp_pos + s_hard + s_allneg - e_hard[s]
        lp = jnp.log(exp_pos / (denom + _EPS) + _EPS)
        loss_blk = loss_blk + jnp.sum(lp)

    @pl.when(i == 0)
    def _():
        out_ref[:, :] = jnp.zeros((1, 1), jnp.float32)

    out_ref[:, :] += loss_blk[None, None]


@jax.jit
def kernel(features, labels):
    batch, dim = features.shape
    blk_r = 2048
    grid = batch // blk_r

    ft = features.T
    lrow = labels.reshape(batch, 1)
    lcol = labels.reshape(1, batch)
    cidx = jnp.arange(batch, dtype=jnp.int32).reshape(1, batch)

    body = functools.partial(_supcon_block_kernel, blk_r=blk_r, batch=batch)
    total = pl.pallas_call(
        body,
        grid=(grid,),
        in_specs=[
            pl.BlockSpec((batch, dim), lambda i: (0, 0)),
            pl.BlockSpec((dim, batch), lambda i: (0, 0)),
            pl.BlockSpec((blk_r, 1), lambda i: (i, 0)),
            pl.BlockSpec((1, batch), lambda i: (0, 0)),
            pl.BlockSpec((1, batch), lambda i: (0, 0)),
        ],
        out_specs=pl.BlockSpec((1, 1), lambda i: (0, 0)),
        out_shape=jax.ShapeDtypeStruct((1, 1), jnp.float32),
    )(features, ft, lrow, lcol, cidx)

    return -total[0, 0] / jnp.float32(batch * _NUM_HARD)
